# baseline (speedup 1.0000x reference)
import functools
import math

import jax
import jax.numpy as jnp
from jax import lax
from jax.experimental import pallas as pl
from jax.experimental.pallas import tpu as pltpu
from jax.experimental.pallas import tpu_sc as plsc

D_MODEL = 64
SCALE = math.sqrt(D_MODEL)

_info = plsc.get_sparse_core_info()
_NC, _NS = _info.num_cores, _info.num_subcores
_NW = _NC * _NS
_BW = 128
_NBUF = 4


@jax.jit
def _embed_lookup(x4, table2):
    st_n, bt_n, _, _ = x4.shape
    n_s = st_n * 8
    assert bt_n == _NW

    mesh = plsc.VectorSubcoreMesh(core_axis_name="c", subcore_axis_name="s")

    @functools.partial(
        pl.kernel,
        mesh=mesh,
        out_type=jax.ShapeDtypeStruct((n_s, 8, _NW, 8, _BW), jnp.float32),
        scratch_types=[
            pltpu.VMEM((st_n, 8, _BW), jnp.int32),
            [pltpu.VMEM((_BW,), jnp.int32)] * _NBUF,
            [pltpu.VMEM((_BW, 2 * D_MODEL), jnp.float32)] * _NBUF,
            [pltpu.VMEM((8, 8, _BW), jnp.float32)] * _NBUF,
            [pltpu.SemaphoreType.DMA] * _NBUF,
            [pltpu.SemaphoreType.DMA] * _NBUF,
        ],
        compiler_params=pltpu.CompilerParams(use_tc_tiling_on_sc=True,
                                               needs_layout_passes=False,
                                               has_side_effects=False,
                                               skip_device_barrier=True),
    )
    def k(x4_hbm, table_hbm, out_hbm, idx_v, idxp, rows, tblk, gsems, osems):
        wid = lax.axis_index("s") * _NC + lax.axis_index("c")
        pltpu.sync_copy(x4_hbm.at[:, wid], idx_v)

        viota = lax.iota(jnp.int32, 16)
        vrots = [lax.rem(viota + t, 16) for t in range(16)]
        drot_hi = [lax.div(vrots[t], 8) for t in range(16)]
        drot_lo = [lax.rem(vrots[t], 8) for t in range(16)]

        def issue_gather(i, b):
            st, si = lax.div(i, 8), lax.rem(i, 8)
            for j in range(8):
                sl = pl.ds(16 * j, 16)
                idxp[b][sl] = lax.shift_right_logical(idx_v[st, si, sl], 1)
            pltpu.async_copy(table_hbm.at[idxp[b]], rows[b], gsems[b])

        def wait_gather(b):
            pltpu.make_async_copy(table_hbm.at[idxp[b]],
                                  rows[b], gsems[b]).wait()

        def issue_out(i, b):
            pltpu.async_copy(tblk[b], out_hbm.at[i, :, wid], osems[b])

        def wait_out(b):
            pltpu.make_async_copy(tblk[b], out_hbm.at[0, :, 0],
                                  osems[b]).wait()

        def transpose_scale(b):
            rb, tb = rows[b], tblk[b]
            @plsc.parallel_loop(0, 32, unroll=2)
            def blk_body(i):
                j, d0 = lax.rem(i, 8), lax.div(i, 8) * 16
                ib = viota + 16 * j
                for t in range(16):
                    idv = vrots[t] + d0
                    v = plsc.load_gather(rb, [ib, idv])
                    plsc.store_scatter(tb, [idv, ib], v * SCALE)

        def transpose_scale(b, i):
            rb, tb = rows[b], tblk[b]
            st, si = lax.div(i, 8), lax.rem(i, 8)
            @plsc.parallel_loop(0, 32, unroll=2)
            def blk_body(p):
                j, dq = lax.rem(p, 8), lax.div(p, 8)
                d0 = dq * 16
                ib = viota + 16 * j
                vidx = idx_v[st, si, pl.ds(16 * j, 16)]
                hd = lax.shift_left(lax.rem(vidx, 2), 6) + d0
                for t in range(16):
                    idt = drot_hi[t] + dq * 2
                    v = plsc.load_gather(rb, [ib, hd + vrots[t]])
                    plsc.store_scatter(tb, [idt, drot_lo[t], ib], v * SCALE)

        def step_b(i, b, drain):
            wait_gather(b)
            if drain:
                wait_out(b)
            transpose_scale(b)
            issue_out(i, b)

        issue_gather(0, 0)
        issue_gather(1, 1)

        def loop_body(kk, carry):
            i0 = 4 * kk
            for m in range(4):
                i = i0 + m
                bp = (m + 2) % 4

                @pl.when(i + 2 < n_s)
                def _():
                    issue_gather(i + 2, bp)

                wait_gather(m)

                @pl.when(i >= 4)
                def _():
                    wait_out(m)

                transpose_scale(m, i)
                issue_out(i, m)
            return carry

        lax.fori_loop(0, n_s // 4, loop_body, 0)
        for b in range(_NBUF):
            wait_out(b)

    return k(x4, table2)


def kernel(x, table):
    b, s = x.shape
    xt = jnp.transpose(x, (1, 0)).astype(jnp.int32)
    x4 = jnp.transpose(xt.reshape(s // 8, 8, b // _BW, _BW), (0, 2, 1, 3))
    out5 = _embed_lookup(x4, table.reshape(table.shape[0] // 2, 2 * D_MODEL))
    out = jnp.transpose(out5, (2, 4, 0, 1, 3)).reshape(b, s, D_MODEL)
    return out


# speedup vs baseline: 2.0495x; 2.0495x over previous
import functools
import math

import jax
import jax.numpy as jnp
from jax import lax
from jax.experimental import pallas as pl
from jax.experimental.pallas import tpu as pltpu
from jax.experimental.pallas import tpu_sc as plsc

D_MODEL = 64
SCALE = math.sqrt(D_MODEL)

_info = plsc.get_sparse_core_info()
_NC, _NS = _info.num_cores, _info.num_subcores
_NW = _NC * _NS
_BW = 128
_NBUF = 4


def _relayout_table(tt, tail2):
    d_dim, v_dim = tt.shape
    n_chunks = v_dim // 128
    tail = v_dim - n_chunks * 128
    per_w = -(-n_chunks // _NW)
    per_w = -4 * (-per_w // 4)

    mesh = plsc.VectorSubcoreMesh(core_axis_name="c", subcore_axis_name="s")

    @functools.partial(
        pl.kernel,
        mesh=mesh,
        out_type=jax.ShapeDtypeStruct((v_dim // 2, 2 * D_MODEL), jnp.float32),
        scratch_types=[
            [pltpu.VMEM((D_MODEL, _BW), jnp.float32)] * _NBUF,
            [pltpu.VMEM((D_MODEL, _BW), jnp.float32)] * _NBUF,
            [pltpu.SemaphoreType.DMA] * _NBUF,
            [pltpu.SemaphoreType.DMA] * _NBUF,
        ],
        compiler_params=pltpu.CompilerParams(use_tc_tiling_on_sc=True,
                                             needs_layout_passes=False),
    )
    def ka(tt_hbm, tail_hbm, out_hbm, abuf, obuf, gsems, osems):
        wid = lax.axis_index("s") * _NC + lax.axis_index("c")

        viota = lax.iota(jnp.int32, 16)
        vrots = [lax.rem(viota + t, 16) for t in range(16)]

        def v0_of(q):
            c = jnp.minimum(wid + _NW * q, n_chunks - 1)
            return pl.multiple_of(c * 128, 128)

        def issue_in(q, b):
            pltpu.async_copy(tt_hbm.at[:, pl.ds(v0_of(q), 128)],
                             abuf[b], gsems[b])

        def wait_in(b):
            pltpu.make_async_copy(tt_hbm.at[:, pl.ds(0, 128)],
                                  abuf[b], gsems[b]).wait()

        def issue_out(q, b):
            pltpu.async_copy(
                obuf[b],
                out_hbm.at[pl.ds(pl.multiple_of(
                    lax.shift_right_logical(v0_of(q), 1), 64), D_MODEL)],
                osems[b])

        def wait_out(b):
            pltpu.make_async_copy(obuf[b],
                                  out_hbm.at[pl.ds(0, D_MODEL)],
                                  osems[b]).wait()

        def transpose_chunk(b):
            ab, ob = abuf[b], obuf[b]
            @plsc.parallel_loop(0, 32, unroll=2)
            def blk_body(p):
                jv, dg = lax.rem(p, 8), lax.div(p, 8)
                ivv = viota + 16 * jv
                pv = lax.shift_right_logical(ivv, 1)
                hv = lax.shift_left(lax.rem(ivv, 2), 6)
                for t in range(16):
                    idd = vrots[t] + dg * 16
                    v = plsc.load_gather(ab, [idd, ivv])
                    plsc.store_scatter(ob, [pv, hv + idd], v)

        issue_in(0, 0)
        issue_in(1, 1)

        def loop_body(kk, carry):
            q0 = 4 * kk
            for m in range(4):
                q = q0 + m

                @pl.when(q + 2 < per_w)
                def _():
                    issue_in(q + 2, (m + 2) % 4)

                wait_in(m)

                @pl.when(q >= 4)
                def _():
                    wait_out(m)

                transpose_chunk(m)
                issue_out(q, m)
            return carry

        lax.fori_loop(0, per_w // 4, loop_body, 0)
        for b in range(_NBUF):
            wait_out(b)

        if tail:
            @pl.when(wid == 0)
            def _():
                pltpu.sync_copy(tail_hbm, obuf[0].at[pl.ds(0, tail // 2)])
                pltpu.sync_copy(
                    obuf[0].at[pl.ds(0, tail // 2)],
                    out_hbm.at[pl.ds(n_chunks * 64, tail // 2)])

    return ka(tt, tail2)


@jax.jit
def _embed_lookup(x4, table):
    st_n, bt_n, _, _ = x4.shape
    n_s = st_n * 8
    assert bt_n == _NW

    v_main = (table.shape[0] // 128) * 128
    table2 = _relayout_table(
        jnp.transpose(table, (1, 0)),
        table[v_main:].reshape(-1, 2 * D_MODEL))

    mesh = plsc.VectorSubcoreMesh(core_axis_name="c", subcore_axis_name="s")

    @functools.partial(
        pl.kernel,
        mesh=mesh,
        out_type=jax.ShapeDtypeStruct((n_s, 8, _NW, 8, _BW), jnp.float32),
        scratch_types=[
            pltpu.VMEM((st_n, 8, _BW), jnp.int32),
            [pltpu.VMEM((_BW,), jnp.int32)] * _NBUF,
            [pltpu.VMEM((_BW, 2 * D_MODEL), jnp.float32)] * _NBUF,
            [pltpu.VMEM((D_MODEL, _BW), jnp.float32)] * _NBUF,
            [pltpu.SemaphoreType.DMA] * _NBUF,
            [pltpu.SemaphoreType.DMA] * _NBUF,
        ],
        compiler_params=pltpu.CompilerParams(use_tc_tiling_on_sc=True,
                                             needs_layout_passes=False),
    )
    def k(x4_hbm, table_hbm, out_hbm, idx_v, idxp, rows, tblk, gsems, osems):
        wid = lax.axis_index("s") * _NC + lax.axis_index("c")
        pltpu.sync_copy(x4_hbm.at[:, wid], idx_v)

        viota = lax.iota(jnp.int32, 16)
        vrots = [lax.rem(viota + t, 16) for t in range(16)]

        def issue_gather(i, b):
            st, si = lax.div(i, 8), lax.rem(i, 8)
            for j in range(8):
                sl = pl.ds(16 * j, 16)
                idxp[b][sl] = lax.shift_right_logical(idx_v[st, si, sl], 1)
            pltpu.async_copy(table_hbm.at[idxp[b]], rows[b], gsems[b])

        def wait_gather(b):
            pltpu.make_async_copy(table_hbm.at[idxp[b]],
                                  rows[b], gsems[b]).wait()

        def issue_out(i, b):
            for dt in range(8):
                pltpu.async_copy(tblk[b].at[pl.ds(8 * dt, 8)],
                                 out_hbm.at[i, dt, wid], osems[b])

        def wait_out(b):
            for dt in range(8):
                pltpu.make_async_copy(tblk[b].at[pl.ds(8 * dt, 8)],
                                      out_hbm.at[0, 0, 0], osems[b]).wait()

        def transpose_scale(b, i):
            rb, tb = rows[b], tblk[b]
            st, si = lax.div(i, 8), lax.rem(i, 8)
            @plsc.parallel_loop(0, 32, unroll=2)
            def blk_body(p):
                j, dq = lax.rem(p, 8), lax.div(p, 8)
                d0 = dq * 16
                ib = viota + 16 * j
                vidx = idx_v[st, si, pl.ds(16 * j, 16)]
                hd = lax.shift_left(lax.rem(vidx, 2), 6) + d0
                for t in range(16):
                    v = plsc.load_gather(rb, [ib, hd + vrots[t]])
                    plsc.store_scatter(tb, [vrots[t] + d0, ib], v * SCALE)

        def step_b(i, b, drain):
            wait_gather(b)
            if drain:
                wait_out(b)
            transpose_scale(b, i)
            issue_out(i, b)

        issue_gather(0, 0)
        issue_gather(1, 1)
        issue_gather(2, 2)
        step_b(0, 0, False)
        issue_gather(3, 3)
        step_b(1, 1, False)
        issue_gather(4, 0)
        step_b(2, 2, False)
        issue_gather(5, 1)
        step_b(3, 3, False)

        def loop_body(kk, carry):
            i0 = 4 * kk
            for m in range(4):
                i = i0 + m
                issue_gather(i + 2, (m + 2) % 4)
                step_b(i, m, True)
            return carry

        lax.fori_loop(1, n_s // 4 - 1, loop_body, 0)

        nl = n_s - 4
        issue_gather(nl + 2, 2)
        step_b(nl + 0, 0, True)
        issue_gather(nl + 3, 3)
        step_b(nl + 1, 1, True)
        step_b(nl + 2, 2, True)
        step_b(nl + 3, 3, True)
        for b in range(_NBUF):
            wait_out(b)

    return k(x4, table2)


def kernel(x, table):
    b, s = x.shape
    xt = jnp.transpose(x, (1, 0)).astype(jnp.int32)
    x4 = jnp.transpose(xt.reshape(s // 8, 8, b // _BW, _BW), (0, 2, 1, 3))
    out5 = _embed_lookup(x4, table)
    out = jnp.transpose(out5, (2, 4, 0, 1, 3)).reshape(b, s, D_MODEL)
    return out
